# baseline (speedup 1.0000x reference)
import jax
import jax.numpy as jnp
from jax.experimental import pallas as pl
from jax.experimental.pallas import tpu as pltpu

INPUT_SIZE = 2048
NUM_EXPERTS = 64
TOP_K = 8
TOKEN_TILE = 2048


def _topk_softmax(logits):
    neg_inf = jnp.float32(-jnp.inf)
    work = logits
    m0 = None
    for _ in range(TOP_K):
        m = jnp.max(work, axis=-1, keepdims=True)
        if m0 is None:
            m0 = m
        work = jnp.where(work == m, neg_inf, work)
    q = jnp.where(logits >= m, logits, neg_inf)
    p = jnp.exp(q - m0)
    den = jnp.sum(p, axis=-1, keepdims=True)
    return p * (1.0 / den)


def _gate_kernel(n_tiles, x_ref, wt_ref, b_ref, out_ref, lg_ref):
    i = pl.program_id(0)
    slot = jax.lax.rem(i, 2)
    logits = jnp.dot(x_ref[...], wt_ref[...], preferred_element_type=jnp.float32)
    routed = _topk_softmax(lg_ref[1 - slot])
    lg_ref[slot] = logits + b_ref[...]
    out_ref[...] = routed


@jax.jit
def kernel(x, W, b):
    B, S, D = x.shape
    tokens = B * S
    x2 = x.reshape(tokens, D)
    wt = W.T
    b2 = b.reshape(1, NUM_EXPERTS)

    n_tiles = tokens // TOKEN_TILE
    import functools

    out = pl.pallas_call(
        functools.partial(_gate_kernel, n_tiles),
        grid=(n_tiles + 1,),
        in_specs=[
            pl.BlockSpec((TOKEN_TILE, D), lambda i: (jnp.minimum(i, n_tiles - 1), 0)),
            pl.BlockSpec((D, NUM_EXPERTS), lambda i: (0, 0)),
            pl.BlockSpec((1, NUM_EXPERTS), lambda i: (0, 0)),
        ],
        out_specs=pl.BlockSpec(
            (TOKEN_TILE, NUM_EXPERTS), lambda i: (jnp.maximum(i - 1, 0), 0)
        ),
        out_shape=jax.ShapeDtypeStruct((tokens, NUM_EXPERTS), jnp.float32),
        scratch_shapes=[pltpu.VMEM((2, TOKEN_TILE, NUM_EXPERTS), jnp.float32)],
    )(x2, wt, b2)
    return out.reshape(B, S, NUM_EXPERTS)


# speedup vs baseline: 1.1280x; 1.1280x over previous
import jax
import jax.numpy as jnp
from jax.experimental import pallas as pl
from jax.experimental.pallas import tpu as pltpu

INPUT_SIZE = 2048
NUM_EXPERTS = 64
TOP_K = 8
TOKEN_TILE = 1024


def _topk_softmax(logits):
    neg_inf = jnp.float32(-jnp.inf)
    work = logits
    m0 = None
    for _ in range(TOP_K):
        m = jnp.max(work, axis=-1, keepdims=True)
        if m0 is None:
            m0 = m
        work = jnp.where(work == m, neg_inf, work)
    q = jnp.where(logits >= m, logits, neg_inf)
    p = jnp.exp(q - m0)
    den = jnp.sum(p, axis=-1, keepdims=True)
    return p * (1.0 / den)


def _gate_kernel(n_tiles, x_ref, wt_ref, b_ref, out_ref, lg_ref):
    i = pl.program_id(0)
    slot = jax.lax.rem(i, 2)
    logits = jax.lax.dot_general(
        x_ref[...],
        wt_ref[...],
        dimension_numbers=(((1,), (1,)), ((), ())),
        preferred_element_type=jnp.float32,
    )
    routed = _topk_softmax(lg_ref[1 - slot])
    lg_ref[slot] = logits + b_ref[...]
    out_ref[...] = routed


@jax.jit
def kernel(x, W, b):
    B, S, D = x.shape
    tokens = B * S
    x2 = x.reshape(tokens, D)
    b2 = b.reshape(1, NUM_EXPERTS)

    n_tiles = tokens // TOKEN_TILE
    import functools

    out = pl.pallas_call(
        functools.partial(_gate_kernel, n_tiles),
        grid=(n_tiles + 1,),
        in_specs=[
            pl.BlockSpec((TOKEN_TILE, D), lambda i: (jnp.minimum(i, n_tiles - 1), 0)),
            pl.BlockSpec((NUM_EXPERTS, D), lambda i: (0, 0)),
            pl.BlockSpec((1, NUM_EXPERTS), lambda i: (0, 0)),
        ],
        out_specs=pl.BlockSpec(
            (TOKEN_TILE, NUM_EXPERTS), lambda i: (jnp.maximum(i - 1, 0), 0)
        ),
        out_shape=jax.ShapeDtypeStruct((tokens, NUM_EXPERTS), jnp.float32),
        scratch_shapes=[pltpu.VMEM((2, TOKEN_TILE, NUM_EXPERTS), jnp.float32)],
    )(x2, W, b2)
    return out.reshape(B, S, NUM_EXPERTS)
